# baseline (speedup 1.0000x reference)
import functools

import jax
import jax.numpy as jnp
from jax import lax
from jax.experimental import pallas as pl
from jax.experimental.pallas import tpu as pltpu
from jax.experimental.pallas import tpu_sc as plsc

N_FIELDS = 26
VOCAB = 100000
EMB = 16
NUM_DENSE = 13
BATCH = 16384

NW = 32
SPW = BATCH // NW
CH = 128
NCH = SPW // CH
TOTAL_CHUNKS = BATCH // CH


def _sc_body(tab_hbm, idx_hbm, dense_hbm, w_hbm, out_hbm,
             idx_v, rows_v, dense_v, w_v, t_v, out_v, sem):
    cid = lax.axis_index("c")
    sid = lax.axis_index("s")
    wid = sid * 2 + cid

    pltpu.sync_copy(w_hbm, w_v)
    half_wemb = w_v[0, :] * 0.5
    w_dense = w_v[1, :]
    bias_vec = w_v[2, :]
    lane = lax.iota(jnp.int32, 16)

    for c in range(NCH):
        chunk = wid * NCH + c
        sample0 = chunk * CH

        pltpu.sync_copy(idx_hbm.at[chunk], idx_v)
        copies = [
            pltpu.async_copy(tab_hbm.at[idx_v.at[f]], rows_v.at[f], sem)
            for f in range(N_FIELDS)
        ]
        pltpu.sync_copy(dense_hbm.at[pl.ds(sample0, CH), :], dense_v)
        for cp in copies:
            cp.wait()

        def group_body(g, _):
            def lane_body(l, _):
                j = g * 16 + l
                e = rows_v[0, j, :]
                s = e
                ss = e * e
                for f in range(1, N_FIELDS):
                    e = rows_v[f, j, :]
                    s = s + e
                    ss = ss + e * e
                t_v[l, :] = (s * s - ss) * half_wemb + dense_v[j, :] * w_dense
                return 0

            lax.fori_loop(0, 16, lane_body, 0)
            red = plsc.load_gather(t_v, [lane, jnp.zeros((16,), jnp.int32)])
            for d in range(1, EMB):
                red = red + plsc.load_gather(
                    t_v, [lane, jnp.full((16,), d, jnp.int32)])
            logits = red + bias_vec
            out_v[pl.ds(g * 16, 16)] = 1.0 / (1.0 + jnp.exp(-logits))
            return 0

        lax.fori_loop(0, CH // 16, group_body, 0)
        pltpu.sync_copy(out_v, out_hbm.at[pl.ds(sample0, CH)])


@functools.partial(jax.jit, static_argnames=())
def kernel(X_sparse, X_dense, tables, dnn_w, dnn_b):
    tab_flat = tables.reshape(N_FIELDS * VOCAB, EMB)
    offs = (jnp.arange(N_FIELDS, dtype=jnp.int32) * VOCAB)[None, :]
    fidx = X_sparse.astype(jnp.int32) + offs
    idx3 = fidx.T.reshape(N_FIELDS, TOTAL_CHUNKS, CH).transpose(1, 0, 2)
    dense16 = jnp.pad(X_dense, ((0, 0), (0, EMB - NUM_DENSE)))
    w_emb = dnn_w[:EMB, 0]
    w_den = jnp.pad(dnn_w[EMB:, 0], (0, EMB - NUM_DENSE))
    b16 = jnp.broadcast_to(dnn_b, (EMB,))
    wcat = jnp.stack([w_emb, w_den, b16])

    call = pl.kernel(
        _sc_body,
        out_type=jax.ShapeDtypeStruct((BATCH,), jnp.float32),
        mesh=plsc.VectorSubcoreMesh(core_axis_name="c", subcore_axis_name="s"),
        compiler_params=pltpu.CompilerParams(
            needs_layout_passes=False, use_tc_tiling_on_sc=False),
        scratch_types=[
            pltpu.VMEM((N_FIELDS, CH), jnp.int32),
            pltpu.VMEM((N_FIELDS, CH, EMB), jnp.float32),
            pltpu.VMEM((CH, EMB), jnp.float32),
            pltpu.VMEM((3, EMB), jnp.float32),
            pltpu.VMEM((16, EMB), jnp.float32),
            pltpu.VMEM((CH,), jnp.float32),
            pltpu.SemaphoreType.DMA,
        ],
    )
    out = call(tab_flat, idx3, dense16, wcat)
    return out.reshape(BATCH, 1)
